# Initial kernel scaffold; baseline (speedup 1.0000x reference)
import functools

import jax
import jax.numpy as jnp
from jax import lax
from jax.experimental import pallas as pl
from jax.experimental.pallas import tpu as pltpu
from jax.experimental.pallas import tpu_sc as plsc

_NUM_WORKERS = 32
_CHUNK = 2048


@functools.cache
def _make_gather(n_idx: int, vocab: int, dim: int, chunk: int):
    b_per_w = n_idx // _NUM_WORKERS
    n_chunks = b_per_w // chunk
    assert n_chunks * chunk == b_per_w

    mesh = plsc.VectorSubcoreMesh(core_axis_name="c", subcore_axis_name="s")

    @functools.partial(
        pl.kernel,
        out_type=jax.ShapeDtypeStruct((n_idx, dim), jnp.float32),
        mesh=mesh,
        scratch_types=[
            pltpu.VMEM((chunk,), jnp.int32),
            pltpu.VMEM((chunk, dim), jnp.float32),
            pltpu.SemaphoreType.DMA,
        ],
    )
    def gather_kernel(idx_hbm, table_hbm, out_hbm, idx_v, rows_v, sem):
        wid = lax.axis_index("s") * 2 + lax.axis_index("c")
        base = wid * b_per_w

        @pl.loop(0, n_chunks)
        def _chunk_body(j):
            off = base + j * chunk
            pltpu.sync_copy(idx_hbm.at[pl.ds(off, chunk)], idx_v)
            pltpu.async_copy(table_hbm.at[idx_v], rows_v, sem).wait()
            pltpu.sync_copy(rows_v, out_hbm.at[pl.ds(off, chunk)])

    return gather_kernel


def kernel(words, table):
    batch, hist = words.shape
    vocab, dim = table.shape
    idx = words.reshape(-1).astype(jnp.int32)
    out = _make_gather(idx.shape[0], vocab, dim, _CHUNK)(idx, table)
    return out.reshape(batch, hist, dim)


# speedup vs baseline: 1.1033x; 1.1033x over previous
import functools

import jax
import jax.numpy as jnp
from jax import lax
from jax.experimental import pallas as pl
from jax.experimental.pallas import tpu as pltpu
from jax.experimental.pallas import tpu_sc as plsc

_NUM_WORKERS = 32
_CHUNK = 1600


@functools.cache
def _make_gather(n_idx: int, vocab: int, dim: int, chunk: int):
    b_per_w = n_idx // _NUM_WORKERS
    n_chunks = b_per_w // chunk
    assert n_chunks * chunk == b_per_w

    mesh = plsc.VectorSubcoreMesh(core_axis_name="c", subcore_axis_name="s")

    @functools.partial(
        pl.kernel,
        out_type=jax.ShapeDtypeStruct((n_idx, dim), jnp.float32),
        mesh=mesh,
        scratch_types=[
            pltpu.VMEM((chunk,), jnp.int32),
            pltpu.VMEM((chunk, dim), jnp.float32),
            pltpu.SemaphoreType.DMA,
        ],
        compiler_params=pltpu.CompilerParams(use_tc_tiling_on_sc=False),
    )
    def gather_kernel(idx_hbm, table_hbm, out_hbm, idx_v, rows_v, sem):
        wid = lax.axis_index("s") * 2 + lax.axis_index("c")
        base = wid * b_per_w

        @pl.loop(0, n_chunks)
        def _chunk_body(j):
            off = base + j * chunk
            pltpu.sync_copy(idx_hbm.at[pl.ds(off, chunk)], idx_v)
            pltpu.async_copy(table_hbm.at[idx_v], rows_v, sem).wait()
            pltpu.sync_copy(rows_v, out_hbm.at[pl.ds(off, chunk)])

    return gather_kernel


def kernel(words, table):
    batch, hist = words.shape
    vocab, dim = table.shape
    idx = words.reshape(-1).astype(jnp.int32)
    out = _make_gather(idx.shape[0], vocab, dim, _CHUNK)(idx, table)
    return out.reshape(batch, hist, dim)


# speedup vs baseline: 1.7050x; 1.5453x over previous
import functools

import jax
import jax.numpy as jnp
from jax import lax
from jax.experimental import pallas as pl
from jax.experimental.pallas import tpu as pltpu
from jax.experimental.pallas import tpu_sc as plsc

_NUM_WORKERS = 32
_BLK = 128
_NBUF = 4


@functools.cache
def _make_gather(hist: int, vocab: int, dim: int, n_btiles: int):
    n_blocks = hist * n_btiles
    blocks_per_w = n_blocks // _NUM_WORKERS
    assert blocks_per_w * _NUM_WORKERS == n_blocks
    assert blocks_per_w % _NBUF == 0
    c_tiles = dim // 8

    mesh = plsc.VectorSubcoreMesh(core_axis_name="c", subcore_axis_name="s")

    @functools.partial(
        pl.kernel,
        out_type=jax.ShapeDtypeStruct((hist, c_tiles, n_btiles, 1024),
                                      jnp.float32),
        mesh=mesh,
        scratch_types=[
            [pltpu.VMEM((_BLK,), jnp.int32)] * _NBUF,
            [pltpu.VMEM((_BLK, dim), jnp.float32)] * _NBUF,
            [pltpu.VMEM((c_tiles * 1024,), jnp.float32)] * _NBUF,
            [pltpu.SemaphoreType.DMA] * _NBUF,
            [pltpu.SemaphoreType.DMA] * _NBUF,
        ],
        compiler_params=pltpu.CompilerParams(use_tc_tiling_on_sc=False,
                                             needs_layout_passes=False),
    )
    def gather_kernel(wordsT_hbm, table_hbm, out_hbm,
                      idx_v, rows_v, trans_v, gsems, ssems):
        wid = lax.axis_index("s") * 2 + lax.axis_index("c")
        base = wid * blocks_per_w
        lanes = lax.iota(jnp.int32, 16)

        def stage(t, b):
            blk = base + t
            h = blk // n_btiles
            b_hi = blk % n_btiles
            pltpu.sync_copy(wordsT_hbm.at[h, pl.ds(b_hi * _BLK, _BLK)],
                            idx_v[b])
            pltpu.async_copy(table_hbm.at[idx_v[b]], rows_v[b], gsems[b])

        def drain_stores(b):
            pltpu.make_async_copy(table_hbm.at[pl.ds(0, _BLK)],
                                  rows_v[b], ssems[b]).wait()

        for b in range(_NBUF):
            stage(b, b)

        @pl.loop(0, blocks_per_w, step=_NBUF)
        def _super(g):
            for b in range(_NBUF):
                t = g + b
                blk = base + t
                h = blk // n_btiles
                b_hi = blk % n_btiles
                pltpu.make_async_copy(table_hbm.at[idx_v[b]], rows_v[b],
                                      gsems[b]).wait()

                @pl.when(g > 0)
                def _free_trans():
                    drain_stores(b)

                @pl.loop(0, _BLK)
                def _transpose(bl):
                    x0 = rows_v[b][bl, pl.ds(0, 16)]
                    x1 = rows_v[b][bl, pl.ds(16, 16)]
                    i0 = lanes * _BLK + bl
                    plsc.store_scatter(trans_v[b], [i0], x0)
                    plsc.store_scatter(trans_v[b], [i0 + 16 * _BLK], x1)

                for c_hi in range(c_tiles):
                    pltpu.async_copy(trans_v[b].at[pl.ds(c_hi * 1024, 1024)],
                                     out_hbm.at[h, c_hi, b_hi], ssems[b])

                @pl.when(g + _NBUF < blocks_per_w)
                def _refill():
                    stage(t + _NBUF, b)

        for b in range(_NBUF):
            drain_stores(b)

    return gather_kernel


def kernel(words, table):
    batch, hist = words.shape
    vocab, dim = table.shape
    wordsT = words.T.astype(jnp.int32)
    n_btiles = batch // _BLK
    out5 = _make_gather(hist, vocab, dim, n_btiles)(wordsT, table)
    out = out5.reshape(hist, dim // 8, n_btiles, 8, _BLK)
    out = out.transpose(2, 4, 0, 1, 3).reshape(batch, hist, dim)
    return out


# speedup vs baseline: 1.8502x; 1.0852x over previous
import functools

import jax
import jax.numpy as jnp
from jax import lax
from jax.experimental import pallas as pl
from jax.experimental.pallas import tpu as pltpu
from jax.experimental.pallas import tpu_sc as plsc

_NUM_WORKERS = 32
_BLK = 128
_NBUF = 4


@functools.cache
def _make_gather(hist: int, vocab: int, dim: int, n_btiles: int):
    n_blocks = hist * n_btiles
    blocks_per_w = n_blocks // _NUM_WORKERS
    assert blocks_per_w * _NUM_WORKERS == n_blocks
    assert blocks_per_w % _NBUF == 0
    c_tiles = dim // 8

    mesh = plsc.VectorSubcoreMesh(core_axis_name="c", subcore_axis_name="s")

    @functools.partial(
        pl.kernel,
        out_type=jax.ShapeDtypeStruct((hist, c_tiles, n_btiles, 1024),
                                      jnp.float32),
        mesh=mesh,
        scratch_types=[
            [pltpu.VMEM((_BLK,), jnp.int32)] * _NBUF,
            [pltpu.VMEM((_BLK, dim), jnp.float32)] * _NBUF,
            [pltpu.VMEM((c_tiles * 1024,), jnp.float32)] * _NBUF,
            [pltpu.SemaphoreType.DMA] * _NBUF,
            [pltpu.SemaphoreType.DMA] * _NBUF,
        ],
        compiler_params=pltpu.CompilerParams(use_tc_tiling_on_sc=False,
                                             needs_layout_passes=False),
    )
    def gather_kernel(wordsT_hbm, table_hbm, out_hbm,
                      idx_v, rows_v, trans_v, gsems, ssems):
        wid = lax.axis_index("s") * 2 + lax.axis_index("c")
        base = wid * blocks_per_w
        lanes = lax.iota(jnp.int32, 16)

        def stage(t, b):
            blk = base + t
            h = blk // n_btiles
            b_hi = blk % n_btiles
            pltpu.sync_copy(wordsT_hbm.at[h, pl.ds(b_hi * _BLK, _BLK)],
                            idx_v[b])
            pltpu.async_copy(table_hbm.at[idx_v[b]], rows_v[b], gsems[b])

        def drain_stores(b):
            pltpu.make_async_copy(table_hbm.at[pl.ds(0, _BLK)],
                                  rows_v[b], ssems[b]).wait()

        for b in range(_NBUF):
            stage(b, b)

        @pl.loop(0, blocks_per_w, step=_NBUF)
        def _super(g):
            for b in range(_NBUF):
                t = g + b
                blk = base + t
                h = blk // n_btiles
                b_hi = blk % n_btiles
                pltpu.make_async_copy(table_hbm.at[idx_v[b]], rows_v[b],
                                      gsems[b]).wait()

                @pl.when(g > 0)
                def _free_trans():
                    drain_stores(b)

                @plsc.parallel_loop(0, _BLK, unroll=8)
                def _transpose(bl):
                    x0 = rows_v[b][bl, pl.ds(0, 16)]
                    x1 = rows_v[b][bl, pl.ds(16, 16)]
                    i0 = lanes * _BLK + bl
                    plsc.store_scatter(trans_v[b], [i0], x0)
                    plsc.store_scatter(trans_v[b], [i0 + 16 * _BLK], x1)

                for c_hi in range(c_tiles):
                    pltpu.async_copy(trans_v[b].at[pl.ds(c_hi * 1024, 1024)],
                                     out_hbm.at[h, c_hi, b_hi], ssems[b])

                @pl.when(g + _NBUF < blocks_per_w)
                def _refill():
                    stage(t + _NBUF, b)

        for b in range(_NBUF):
            drain_stores(b)

    return gather_kernel


def kernel(words, table):
    batch, hist = words.shape
    vocab, dim = table.shape
    wordsT = words.T.astype(jnp.int32)
    n_btiles = batch // _BLK
    out5 = _make_gather(hist, vocab, dim, n_btiles)(wordsT, table)
    out = out5.reshape(hist, dim // 8, n_btiles, 8, _BLK)
    out = out.transpose(2, 4, 0, 1, 3).reshape(batch, hist, dim)
    return out


# speedup vs baseline: 2.7824x; 1.5038x over previous
import functools

import jax
import jax.numpy as jnp
from jax import lax
from jax.experimental import pallas as pl
from jax.experimental.pallas import tpu as pltpu
from jax.experimental.pallas import tpu_sc as plsc

_NUM_WORKERS = 32
_BLK = 128
_NBUF = 4
_PITCH = 129


@functools.cache
def _make_gather(hist: int, vocab: int, dim: int, n_btiles: int):
    n_blocks = hist * n_btiles
    blocks_per_w = n_blocks // _NUM_WORKERS
    assert blocks_per_w * _NUM_WORKERS == n_blocks
    assert blocks_per_w % _NBUF == 0
    c_tiles = dim // 8

    mesh = plsc.VectorSubcoreMesh(core_axis_name="c", subcore_axis_name="s")

    @functools.partial(
        pl.kernel,
        out_type=jax.ShapeDtypeStruct((hist, c_tiles, n_btiles, 1024),
                                      jnp.float32),
        mesh=mesh,
        scratch_types=[
            [pltpu.VMEM((_BLK,), jnp.int32)] * _NBUF,
            [pltpu.VMEM((_BLK, dim), jnp.float32)] * _NBUF,
            [pltpu.VMEM((dim, _PITCH), jnp.float32)] * _NBUF,
            [pltpu.SemaphoreType.DMA] * _NBUF,
            [pltpu.SemaphoreType.DMA] * _NBUF,
            [pltpu.SemaphoreType.DMA] * _NBUF,
        ],
        compiler_params=pltpu.CompilerParams(use_tc_tiling_on_sc=False,
                                             needs_layout_passes=False),
    )
    def gather_kernel(wordsT_hbm, table_hbm, out_hbm,
                      idx_v, rows_v, trans_v, isems, gsems, ssems):
        wid = lax.axis_index("s") * 2 + lax.axis_index("c")
        base = wid * blocks_per_w
        lanes = lax.iota(jnp.int32, 16)

        def idx_src(t):
            blk = base + t
            h = blk // n_btiles
            b_hi = blk % n_btiles
            return wordsT_hbm.at[h, pl.ds(b_hi * _BLK, _BLK)]

        def drain_stores(b):
            pltpu.make_async_copy(table_hbm.at[pl.ds(0, _BLK)],
                                  rows_v[b], ssems[b]).wait()

        for b in range(_NBUF):
            pltpu.sync_copy(idx_src(b), idx_v[b])
            pltpu.async_copy(table_hbm.at[idx_v[b]], rows_v[b], gsems[b])

        @pl.loop(0, blocks_per_w, step=_NBUF)
        def _super(g):
            for b in range(_NBUF):
                t = g + b
                blk = base + t
                h = blk // n_btiles
                b_hi = blk % n_btiles
                pltpu.make_async_copy(table_hbm.at[idx_v[b]], rows_v[b],
                                      gsems[b]).wait()

                @pl.when(g > 0)
                def _free_trans():
                    drain_stores(b)

                @plsc.parallel_loop(0, _BLK, unroll=8)
                def _transpose(bl):
                    bvec = jnp.full((16,), bl, jnp.int32)
                    x0 = rows_v[b][bl, pl.ds(0, 16)]
                    x1 = rows_v[b][bl, pl.ds(16, 16)]
                    plsc.store_scatter(trans_v[b], [lanes, bvec], x0)
                    plsc.store_scatter(trans_v[b], [lanes + 16, bvec], x1)

                for c in range(dim):
                    pltpu.async_copy(
                        trans_v[b].at[c, pl.ds(0, _BLK)],
                        out_hbm.at[h, c // 8, b_hi,
                                   pl.ds((c % 8) * _BLK, _BLK)],
                        ssems[b])

                @pl.when(g + _NBUF < blocks_per_w)
                def _refill():
                    pltpu.async_copy(idx_src(t + _NBUF), idx_v[b], isems[b])
                    pltpu.make_async_copy(idx_src(t + _NBUF), idx_v[b],
                                          isems[b]).wait()
                    pltpu.async_copy(table_hbm.at[idx_v[b]], rows_v[b],
                                     gsems[b])

        for b in range(_NBUF):
            drain_stores(b)

    return gather_kernel


def kernel(words, table):
    batch, hist = words.shape
    vocab, dim = table.shape
    wordsT = words.T.astype(jnp.int32)
    n_btiles = batch // _BLK
    out5 = _make_gather(hist, vocab, dim, n_btiles)(wordsT, table)
    out = out5.reshape(hist, dim // 8, n_btiles, 8, _BLK)
    out = out.transpose(2, 4, 0, 1, 3).reshape(batch, hist, dim)
    return out
